# Initial kernel scaffold; baseline (speedup 1.0000x reference)
import jax
import jax.numpy as jnp
from jax.experimental import pallas as pl


def kernel(flat1, cu_seqlens1, flat2, cu_seqlens2):
    raise NotImplementedError("write your pallas kernel here")


# speedup vs baseline: 1.0260x; 1.0260x over previous
import functools

import jax
import jax.numpy as jnp
from jax import lax
from jax.experimental import pallas as pl
from jax.experimental.pallas import tpu as pltpu
from jax.experimental.pallas import tpu_sc as plsc

B = 16
TOTAL = 4096
SEQ_LEN = 128
CLS_ID = 101
SEP_ID = 102
HALF = SEQ_LEN // 2


def _extract_splat(vec, lane_splat):
    return vec.at[lane_splat].get(mode="promise_in_bounds")


def _body(flat1_hbm, c1lo_hbm, c1hi_hbm, flat2_hbm, c2lo_hbm, c2hi_hbm,
          ow_hbm, om_hbm, ot_hbm,
          f1_v, f2_v, c1lo_v, c1hi_v, c2lo_v, c2hi_v, bw_v, bm_v, bt_v):
    c = lax.axis_index("c")
    s = lax.axis_index("s")
    wid = s * 2 + c

    pltpu.sync_copy(flat1_hbm, f1_v)
    pltpu.sync_copy(flat2_hbm, f2_v)
    pltpu.sync_copy(c1lo_hbm, c1lo_v)
    pltpu.sync_copy(c1hi_hbm, c1hi_v)
    pltpu.sync_copy(c2lo_hbm, c2lo_v)
    pltpu.sync_copy(c2hi_hbm, c2hi_v)

    c1lo = c1lo_v[...]
    c1hi = c1hi_v[...]
    c2lo = c2lo_v[...]
    c2hi = c2hi_v[...]

    len1 = c1hi - c1lo
    len2 = c2hi - c2lo
    avail = SEQ_LEN - 3
    cap1 = (avail + 1) // 2
    l1v = jnp.minimum(len1, jnp.maximum(cap1, avail - len2))
    l2v = jnp.minimum(len2, avail - l1v)

    lanes = lax.iota(jnp.int32, 16)
    lane_splat = jnp.full((16,), s, jnp.int32)
    s1 = _extract_splat(c1lo, lane_splat)
    s2 = _extract_splat(c2lo, lane_splat)
    l1 = _extract_splat(l1v, lane_splat)
    l2 = _extract_splat(l2v, lane_splat)
    end = l1 + l2 + 2

    base = c * HALF
    for j in range(HALF // 16):
        idx = base + j * 16 + lanes
        g1 = jnp.clip(s1 + idx - 1, 0, TOTAL - 1)
        g2 = jnp.clip(s2 + (idx - l1 - 2), 0, TOTAL - 1)
        tok1 = plsc.load_gather(f1_v, [g1])
        tok2 = plsc.load_gather(f2_v, [g2])

        in1 = (idx >= 1) & (idx <= l1)
        in2 = (idx >= l1 + 2) & (idx <= end)
        is_sep = (idx == l1 + 1) | (idx == end)

        word = jnp.where(idx == 0, CLS_ID, 0)
        word = jnp.where(in1, tok1, word)
        word = jnp.where(in2, tok2, word)
        word = jnp.where(is_sep, SEP_ID, word)

        sl = pl.ds(j * 16, 16)
        bw_v[sl] = word
        bm_v[sl] = (idx <= end).astype(jnp.int32)
        bt_v[sl] = in2.astype(jnp.int32)

    pltpu.sync_copy(bw_v, ow_hbm.at[wid])
    pltpu.sync_copy(bm_v, om_hbm.at[wid])
    pltpu.sync_copy(bt_v, ot_hbm.at[wid])


@jax.jit
def kernel(flat1, cu_seqlens1, flat2, cu_seqlens2):
    c1lo = cu_seqlens1[:-1]
    c1hi = cu_seqlens1[1:]
    c2lo = cu_seqlens2[:-1]
    c2hi = cu_seqlens2[1:]

    mesh = plsc.VectorSubcoreMesh(core_axis_name="c", subcore_axis_name="s")
    run = functools.partial(
        pl.kernel,
        out_type=[jax.ShapeDtypeStruct((2 * B, HALF), jnp.int32)] * 3,
        mesh=mesh,
        compiler_params=pltpu.CompilerParams(needs_layout_passes=False),
        scratch_types=[
            pltpu.VMEM((TOTAL,), jnp.int32),
            pltpu.VMEM((TOTAL,), jnp.int32),
            pltpu.VMEM((B,), jnp.int32),
            pltpu.VMEM((B,), jnp.int32),
            pltpu.VMEM((B,), jnp.int32),
            pltpu.VMEM((B,), jnp.int32),
            pltpu.VMEM((HALF,), jnp.int32),
            pltpu.VMEM((HALF,), jnp.int32),
            pltpu.VMEM((HALF,), jnp.int32),
        ],
    )(_body)
    ow, om, ot = run(flat1, c1lo, c1hi, flat2, c2lo, c2hi)
    return (ow.reshape(B, SEQ_LEN), om.reshape(B, SEQ_LEN),
            ot.reshape(B, SEQ_LEN))


# speedup vs baseline: 1.1237x; 1.0953x over previous
import functools

import jax
import jax.numpy as jnp
from jax import lax
from jax.experimental import pallas as pl
from jax.experimental.pallas import tpu as pltpu
from jax.experimental.pallas import tpu_sc as plsc

B = 16
TOTAL = 4096
SEQ_LEN = 128
CLS_ID = 101
SEP_ID = 102
HALF = SEQ_LEN // 2


def _extract_splat(vec, lane_splat):
    return vec.at[lane_splat].get(mode="promise_in_bounds")


def _body(flat1_hbm, flat2_hbm, cu_hbm,
          ow_hbm, om_hbm, ot_hbm,
          f1_v, f2_v, cu_v, bw_v, bm_v, bt_v, sem_in, sem_out):
    c = lax.axis_index("c")
    s = lax.axis_index("s")
    wid = s * 2 + c

    cp1 = pltpu.make_async_copy(flat1_hbm, f1_v, sem_in)
    cp2 = pltpu.make_async_copy(flat2_hbm, f2_v, sem_in)
    cp3 = pltpu.make_async_copy(cu_hbm, cu_v, sem_in)
    cp1.start()
    cp2.start()
    cp3.start()
    cp3.wait()
    c1lo = cu_v[pl.ds(0, 16)]
    c1hi = cu_v[pl.ds(16, 16)]
    c2lo = cu_v[pl.ds(32, 16)]
    c2hi = cu_v[pl.ds(48, 16)]

    len1 = c1hi - c1lo
    len2 = c2hi - c2lo
    avail = SEQ_LEN - 3
    cap1 = (avail + 1) // 2
    l1v = jnp.minimum(len1, jnp.maximum(cap1, avail - len2))
    l2v = jnp.minimum(len2, avail - l1v)

    lanes = lax.iota(jnp.int32, 16)
    lane_splat = jnp.full((16,), s, jnp.int32)
    s1 = _extract_splat(c1lo, lane_splat)
    s2 = _extract_splat(c2lo, lane_splat)
    l1 = _extract_splat(l1v, lane_splat)
    l2 = _extract_splat(l2v, lane_splat)
    end = l1 + l2 + 2

    cp1.wait()
    cp2.wait()

    base = c * HALF
    for j in range(HALF // 16):
        idx = base + j * 16 + lanes
        g1 = jnp.clip(s1 + idx - 1, 0, TOTAL - 1)
        g2 = jnp.clip(s2 + (idx - l1 - 2), 0, TOTAL - 1)
        tok1 = plsc.load_gather(f1_v, [g1])
        tok2 = plsc.load_gather(f2_v, [g2])

        in1 = (idx >= 1) & (idx <= l1)
        in2 = (idx >= l1 + 2) & (idx <= end)
        is_sep = (idx == l1 + 1) | (idx == end)

        word = jnp.where(idx == 0, CLS_ID, 0)
        word = jnp.where(in1, tok1, word)
        word = jnp.where(in2, tok2, word)
        word = jnp.where(is_sep, SEP_ID, word)

        sl = pl.ds(j * 16, 16)
        bw_v[sl] = word
        bm_v[sl] = (idx <= end).astype(jnp.int32)
        bt_v[sl] = in2.astype(jnp.int32)

    ocp1 = pltpu.make_async_copy(bw_v, ow_hbm.at[wid], sem_out)
    ocp2 = pltpu.make_async_copy(bm_v, om_hbm.at[wid], sem_out)
    ocp3 = pltpu.make_async_copy(bt_v, ot_hbm.at[wid], sem_out)
    ocp1.start()
    ocp2.start()
    ocp3.start()
    ocp1.wait()
    ocp2.wait()
    ocp3.wait()


@jax.jit
def kernel(flat1, cu_seqlens1, flat2, cu_seqlens2):
    cu = jnp.concatenate([cu_seqlens1[:-1], cu_seqlens1[1:],
                          cu_seqlens2[:-1], cu_seqlens2[1:]])

    mesh = plsc.VectorSubcoreMesh(core_axis_name="c", subcore_axis_name="s")
    run = functools.partial(
        pl.kernel,
        out_type=[jax.ShapeDtypeStruct((2 * B, HALF), jnp.int32)] * 3,
        mesh=mesh,
        compiler_params=pltpu.CompilerParams(needs_layout_passes=False),
        scratch_types=[
            pltpu.VMEM((TOTAL,), jnp.int32),
            pltpu.VMEM((TOTAL,), jnp.int32),
            pltpu.VMEM((4 * B,), jnp.int32),
            pltpu.VMEM((HALF,), jnp.int32),
            pltpu.VMEM((HALF,), jnp.int32),
            pltpu.VMEM((HALF,), jnp.int32),
            pltpu.SemaphoreType.DMA,
            pltpu.SemaphoreType.DMA,
        ],
    )(_body)
    ow, om, ot = run(flat1, flat2, cu)
    return (ow.reshape(B, SEQ_LEN), om.reshape(B, SEQ_LEN),
            ot.reshape(B, SEQ_LEN))
